# baseline (speedup 1.0000x reference)
import jax
import jax.numpy as jnp
from jax import lax
from jax.experimental import pallas as pl
from jax.experimental.pallas import tpu as pltpu
from jax.experimental.pallas import tpu_sc as plsc

N = 10000
E = 320000
D = 128
DH = D // 2

NC = 2
NS = 16
EPS = E // NS
EPSP = 20480
CH = 80
NCHUNK = EPSP // CH
NBUF = 2
NP = 10240
RPZ = NP // NS

BN = 2048
GRID = NP // BN


def _dot_t(a, w):
    return lax.dot_general(a, w, (((1,), (1,)), ((), ())),
                           preferred_element_type=jnp.float32)


def _pre_body(x_ref, wl_ref, wr_ref, b_ref, ya_ref, yb_ref, r_ref):
    xb = x_ref[...]
    y = _dot_t(xb, wl_ref[...])
    ya_ref[...] = y[:, :DH]
    yb_ref[...] = y[:, DH:]
    r_ref[...] = _dot_t(xb, wr_ref[...]) + b_ref[...]


def _tc_pre(x, wl, wr, b):
    return pl.pallas_call(
        _pre_body,
        grid=(GRID,),
        in_specs=[
            pl.BlockSpec((BN, D), lambda i: (i, 0)),
            pl.BlockSpec((D, D), lambda i: (0, 0)),
            pl.BlockSpec((D, D), lambda i: (0, 0)),
            pl.BlockSpec((1, D), lambda i: (0, 0)),
        ],
        out_specs=[
            pl.BlockSpec((BN, DH), lambda i: (i, 0)),
            pl.BlockSpec((BN, DH), lambda i: (i, 0)),
            pl.BlockSpec((BN, D), lambda i: (i, 0)),
        ],
        out_shape=[
            jax.ShapeDtypeStruct((N, DH), jnp.float32),
            jax.ShapeDtypeStruct((N, DH), jnp.float32),
            jax.ShapeDtypeStruct((N, D), jnp.float32),
        ],
    )(x, wl, wr, b)


def _combine(s_ref, cnt_ref, r_ref):
    summed = jnp.concatenate([s_ref[0], s_ref[1]], axis=1)
    inv = 1.0 / jnp.maximum(cnt_ref[0, :], 1.0)
    return summed * inv[:, None] + r_ref[...]


def _combine_pre_body(s_ref, cnt_ref, r_ref, wl_ref, wr_ref, b_ref,
                      ya_ref, yb_ref, r2_ref):
    h = jnp.maximum(_combine(s_ref, cnt_ref, r_ref), 0.0)
    y = _dot_t(h, wl_ref[...])
    ya_ref[...] = y[:, :DH]
    yb_ref[...] = y[:, DH:]
    r2_ref[...] = _dot_t(h, wr_ref[...]) + b_ref[...]


def _tc_combine_pre(s, cnt, r, wl, wr, b):
    return pl.pallas_call(
        _combine_pre_body,
        grid=(GRID,),
        in_specs=[
            pl.BlockSpec((NC, BN, DH), lambda i: (0, i, 0)),
            pl.BlockSpec((8, BN), lambda i: (0, i)),
            pl.BlockSpec((BN, D), lambda i: (i, 0)),
            pl.BlockSpec((D, D), lambda i: (0, 0)),
            pl.BlockSpec((D, D), lambda i: (0, 0)),
            pl.BlockSpec((1, D), lambda i: (0, 0)),
        ],
        out_specs=[
            pl.BlockSpec((BN, DH), lambda i: (i, 0)),
            pl.BlockSpec((BN, DH), lambda i: (i, 0)),
            pl.BlockSpec((BN, D), lambda i: (i, 0)),
        ],
        out_shape=[
            jax.ShapeDtypeStruct((N, DH), jnp.float32),
            jax.ShapeDtypeStruct((N, DH), jnp.float32),
            jax.ShapeDtypeStruct((N, D), jnp.float32),
        ],
    )(s, cnt, r, wl, wr, b)


def _final_body(s_ref, cnt_ref, r_ref, o_ref):
    o_ref[...] = _combine(s_ref, cnt_ref, r_ref)


def _tc_final(s, cnt, r):
    return pl.pallas_call(
        _final_body,
        grid=(GRID,),
        in_specs=[
            pl.BlockSpec((NC, BN, DH), lambda i: (0, i, 0)),
            pl.BlockSpec((8, BN), lambda i: (0, i)),
            pl.BlockSpec((BN, D), lambda i: (i, 0)),
        ],
        out_specs=pl.BlockSpec((BN, D), lambda i: (i, 0)),
        out_shape=jax.ShapeDtypeStruct((N, D), jnp.float32),
    )(s, cnt, r)


def _make_segsum(with_counts):
    mesh = plsc.VectorSubcoreMesh(core_axis_name="c", subcore_axis_name="s",
                                  num_cores=NC, num_subcores=NS)
    out_type = [jax.ShapeDtypeStruct((NC, NP, DH), jnp.float32)]
    if with_counts:
        out_type.append(jax.ShapeDtypeStruct((8, NP), jnp.float32))
    scratch = (
        [pltpu.VMEM((EPSP,), jnp.int32)]
        + [pltpu.VMEM((CH, DH), jnp.float32)] * NBUF
        + [pltpu.VMEM((CH,), jnp.int32)] * NBUF
        + [
            pltpu.VMEM((CH,), jnp.float32),
            pltpu.VMEM_SHARED((NP, DH), jnp.float32),
            pltpu.VMEM_SHARED((NP,), jnp.float32),
        ]
        + [pltpu.SemaphoreType.DMA] * (4 * NBUF)
    )

    def body(ya_hbm, yb_hbm, src_hbm, dst_hbm, zf_hbm, zc_hbm, *rest):
        if with_counts:
            (s_out, cnt_out, srca, *rest2) = rest
        else:
            (s_out, srca, *rest2) = rest
            cnt_out = None
        rows = tuple(rest2[:NBUF])
        dstb = tuple(rest2[NBUF:2 * NBUF])
        ones, acc, cacc = rest2[2 * NBUF:2 * NBUF + 3]
        sems = rest2[2 * NBUF + 3:]
        gsem = tuple(sems[:NBUF])
        ssem = tuple(sems[NBUF:2 * NBUF])
        csem = tuple(sems[2 * NBUF:3 * NBUF])
        isem = tuple(sems[3 * NBUF:4 * NBUF])
        c = lax.axis_index("c")
        s = lax.axis_index("s")
        pltpu.sync_copy(zf_hbm.at[pl.ds(s * RPZ, RPZ)],
                        acc.at[pl.ds(s * RPZ, RPZ)])
        base = s * EPSP
        pltpu.sync_copy(src_hbm.at[pl.ds(base, EPSP)], srca)
        for b in range(NBUF):
            pltpu.sync_copy(dst_hbm.at[pl.ds(base + b * CH, CH)], dstb[b])
        if with_counts:
            @pl.when(jnp.logical_and(c == 0, s == 0))
            def _():
                pltpu.sync_copy(zc_hbm, cacc)

            def fill(i, carry):
                ones[pl.ds(i * 16, 16)] = jnp.ones((16,), jnp.float32)
                return carry
            lax.fori_loop(0, CH // 16, fill, 0)
        plsc.subcore_barrier()

        def issue_gather(t, b):
            idx = srca.at[pl.ds(t * CH, CH)]

            @pl.when(c == 0)
            def _():
                pltpu.async_copy(ya_hbm.at[idx], rows[b], gsem[b])

            @pl.when(c == 1)
            def _():
                pltpu.async_copy(yb_hbm.at[idx], rows[b], gsem[b])

        def drain_gather(b):
            pltpu.make_async_copy(ya_hbm.at[pl.ds(0, CH)], rows[b],
                                  gsem[b]).wait()

        def drain_scatter(b):
            pltpu.make_async_copy(rows[b], acc.at[pl.ds(0, CH)],
                                  ssem[b]).wait()

        def drain_count(b):
            pltpu.make_async_copy(ones, cacc.at[pl.ds(0, CH)],
                                  csem[b]).wait()

        def drain_idx(b):
            pltpu.make_async_copy(dst_hbm.at[pl.ds(0, CH)], dstb[b],
                                  isem[b]).wait()

        for b in range(NBUF):
            issue_gather(b, b)

        def outer(i, carry):
            for b in range(NBUF):
                t = i * NBUF + b
                b1 = (b + NBUF - 1) % NBUF
                b2 = (b + 2) % NBUF
                drain_gather(b)

                @pl.when(t >= NBUF)
                def _():
                    drain_idx(b)

                pltpu.sync_copy(rows[b], acc.at[dstb[b]], add=True)
                if with_counts:
                    @pl.when(c == 0)
                    def _():
                        pltpu.sync_copy(ones, cacc.at[dstb[b]], add=True)

                @pl.when(jnp.logical_and(t >= NBUF - 2, t + 2 < NCHUNK))
                def _():
                    pltpu.async_copy(
                        dst_hbm.at[pl.ds(base + (t + 2) * CH, CH)],
                        dstb[b2], isem[b2])
                    issue_gather(t + 2, b2)
            return carry
        lax.fori_loop(0, NCHUNK // NBUF, outer, 0)
        plsc.subcore_barrier()

        pltpu.sync_copy(acc.at[pl.ds(s * RPZ, RPZ)],
                        s_out.at[c, pl.ds(s * RPZ, RPZ)])
        if with_counts:
            @pl.when(jnp.logical_and(c == 0, s == 0))
            def _():
                pltpu.sync_copy(cacc, cnt_out.at[0])

    return pl.kernel(body, out_type=out_type, mesh=mesh,
                     scratch_types=scratch,
                     compiler_params=pltpu.CompilerParams(
                         use_tc_tiling_on_sc=False))


_segsum_counts = _make_segsum(True)
_segsum_plain = _make_segsum(False)


def kernel(x, edge_index, W1_l, b1_l, W1_r, W2_l, b2_l, W2_r):
    pad = EPSP - EPS
    trash = N + (jnp.arange(pad, dtype=jnp.int32) % (NP - N))
    trash = jnp.broadcast_to(trash, (NS, pad))
    src = jnp.pad(edge_index[0].reshape(NS, EPS), ((0, 0), (0, pad))).reshape(-1)
    dst = jnp.concatenate([edge_index[1].reshape(NS, EPS), trash],
                          axis=1).reshape(-1)
    zf = jnp.zeros((NP, DH), jnp.float32)
    zc = jnp.zeros((NP,), jnp.float32)

    y1a, y1b, r1 = _tc_pre(x, W1_l, W1_r, b1_l.reshape(1, D))
    s1, cnt = _segsum_counts(y1a, y1b, src, dst, zf, zc)
    y2a, y2b, r2 = _tc_combine_pre(s1, cnt, r1, W2_l, W2_r, b2_l.reshape(1, D))
    (s2,) = _segsum_plain(y2a, y2b, src, dst, zf, zc)
    return _tc_final(s2, cnt, r2)


# speedup vs baseline: 2.0428x; 2.0428x over previous
import jax
import jax.numpy as jnp
from jax import lax
from jax.experimental import pallas as pl
from jax.experimental.pallas import tpu as pltpu
from jax.experimental.pallas import tpu_sc as plsc

N = 10000
E = 320000
D = 128
DH = D // 2

NC = 2
NS = 16
EPS = E // NS
CH = 80
NCHUNK = EPS // CH
NBUF = 4
NMAIN = (NCHUNK // NBUF) * NBUF
NP = 10240
RPZ = NP // NS

BN = 2048
GRID = NP // BN


def _dot_t(a, w):
    return lax.dot_general(a, w, (((1,), (1,)), ((), ())),
                           preferred_element_type=jnp.float32)


def _pre_body(x_ref, wl_ref, wr_ref, b_ref, ya_ref, yb_ref, r_ref):
    xb = x_ref[...]
    y = _dot_t(xb, wl_ref[...])
    ya_ref[...] = y[:, :DH]
    yb_ref[...] = y[:, DH:]
    r_ref[...] = _dot_t(xb, wr_ref[...]) + b_ref[...]


def _tc_pre(x, wl, wr, b):
    return pl.pallas_call(
        _pre_body,
        grid=(GRID,),
        in_specs=[
            pl.BlockSpec((BN, D), lambda i: (i, 0)),
            pl.BlockSpec((D, D), lambda i: (0, 0)),
            pl.BlockSpec((D, D), lambda i: (0, 0)),
            pl.BlockSpec((1, D), lambda i: (0, 0)),
        ],
        out_specs=[
            pl.BlockSpec((BN, DH), lambda i: (i, 0)),
            pl.BlockSpec((BN, DH), lambda i: (i, 0)),
            pl.BlockSpec((BN, D), lambda i: (i, 0)),
        ],
        out_shape=[
            jax.ShapeDtypeStruct((N, DH), jnp.float32),
            jax.ShapeDtypeStruct((N, DH), jnp.float32),
            jax.ShapeDtypeStruct((N, D), jnp.float32),
        ],
    )(x, wl, wr, b)


def _combine(s_ref, cnt_ref, r_ref):
    summed = jnp.concatenate([s_ref[0], s_ref[1]], axis=1)
    inv = 1.0 / jnp.maximum(cnt_ref[0, :], 1.0)
    return summed * inv[:, None] + r_ref[...]


def _combine_pre_body(s_ref, cnt_ref, r_ref, wl_ref, wr_ref, b_ref,
                      ya_ref, yb_ref, r2_ref):
    h = jnp.maximum(_combine(s_ref, cnt_ref, r_ref), 0.0)
    y = _dot_t(h, wl_ref[...])
    ya_ref[...] = y[:, :DH]
    yb_ref[...] = y[:, DH:]
    r2_ref[...] = _dot_t(h, wr_ref[...]) + b_ref[...]


def _tc_combine_pre(s, cnt, r, wl, wr, b):
    return pl.pallas_call(
        _combine_pre_body,
        grid=(GRID,),
        in_specs=[
            pl.BlockSpec((NC, BN, DH), lambda i: (0, i, 0)),
            pl.BlockSpec((8, BN), lambda i: (0, i)),
            pl.BlockSpec((BN, D), lambda i: (i, 0)),
            pl.BlockSpec((D, D), lambda i: (0, 0)),
            pl.BlockSpec((D, D), lambda i: (0, 0)),
            pl.BlockSpec((1, D), lambda i: (0, 0)),
        ],
        out_specs=[
            pl.BlockSpec((BN, DH), lambda i: (i, 0)),
            pl.BlockSpec((BN, DH), lambda i: (i, 0)),
            pl.BlockSpec((BN, D), lambda i: (i, 0)),
        ],
        out_shape=[
            jax.ShapeDtypeStruct((N, DH), jnp.float32),
            jax.ShapeDtypeStruct((N, DH), jnp.float32),
            jax.ShapeDtypeStruct((N, D), jnp.float32),
        ],
    )(s, cnt, r, wl, wr, b)


def _final_body(s_ref, cnt_ref, r_ref, o_ref):
    o_ref[...] = _combine(s_ref, cnt_ref, r_ref)


def _tc_final(s, cnt, r):
    return pl.pallas_call(
        _final_body,
        grid=(GRID,),
        in_specs=[
            pl.BlockSpec((NC, BN, DH), lambda i: (0, i, 0)),
            pl.BlockSpec((8, BN), lambda i: (0, i)),
            pl.BlockSpec((BN, D), lambda i: (i, 0)),
        ],
        out_specs=pl.BlockSpec((BN, D), lambda i: (i, 0)),
        out_shape=jax.ShapeDtypeStruct((N, D), jnp.float32),
    )(s, cnt, r)


def _make_segsum(with_counts):
    mesh = plsc.VectorSubcoreMesh(core_axis_name="c", subcore_axis_name="s",
                                  num_cores=NC, num_subcores=NS)
    out_type = [jax.ShapeDtypeStruct((NC, NP, DH), jnp.float32)]
    if with_counts:
        out_type.append(jax.ShapeDtypeStruct((8, NP), jnp.float32))
    scratch = (
        [pltpu.VMEM((EPS,), jnp.int32)]
        + [pltpu.VMEM((CH, DH), jnp.float32)] * NBUF
        + [pltpu.VMEM((CH,), jnp.int32)] * NBUF
        + [
            pltpu.VMEM((CH,), jnp.float32),
            pltpu.VMEM_SHARED((NP, DH), jnp.float32),
            pltpu.VMEM_SHARED((NP,), jnp.float32),
        ]
        + [pltpu.SemaphoreType.DMA] * (4 * NBUF)
    )

    def body(ya_hbm, yb_hbm, src_hbm, dst_hbm, zf_hbm, zc_hbm, *rest):
        if with_counts:
            (s_out, cnt_out, srca, *rest2) = rest
        else:
            (s_out, srca, *rest2) = rest
            cnt_out = None
        rows = tuple(rest2[:NBUF])
        dstb = tuple(rest2[NBUF:2 * NBUF])
        ones, acc, cacc = rest2[2 * NBUF:2 * NBUF + 3]
        sems = rest2[2 * NBUF + 3:]
        gsem = tuple(sems[:NBUF])
        ssem = tuple(sems[NBUF:2 * NBUF])
        csem = tuple(sems[2 * NBUF:3 * NBUF])
        isem = tuple(sems[3 * NBUF:4 * NBUF])
        c = lax.axis_index("c")
        s = lax.axis_index("s")
        pltpu.sync_copy(zf_hbm.at[pl.ds(s * RPZ, RPZ)],
                        acc.at[pl.ds(s * RPZ, RPZ)])
        base = s * EPS
        pltpu.sync_copy(src_hbm.at[pl.ds(base, EPS)], srca)
        for b in range(NBUF):
            pltpu.sync_copy(dst_hbm.at[pl.ds(base + b * CH, CH)], dstb[b])
        if with_counts:
            @pl.when(jnp.logical_and(c == 0, s == 0))
            def _():
                pltpu.sync_copy(zc_hbm, cacc)

            def fill(i, carry):
                ones[pl.ds(i * 16, 16)] = jnp.ones((16,), jnp.float32)
                return carry
            lax.fori_loop(0, CH // 16, fill, 0)
        plsc.subcore_barrier()

        def issue_gather(t, b):
            idx = srca.at[pl.ds(t * CH, CH)]

            @pl.when(c == 0)
            def _():
                pltpu.async_copy(ya_hbm.at[idx], rows[b], gsem[b])

            @pl.when(c == 1)
            def _():
                pltpu.async_copy(yb_hbm.at[idx], rows[b], gsem[b])

        def drain_gather(b):
            pltpu.make_async_copy(ya_hbm.at[pl.ds(0, CH)], rows[b],
                                  gsem[b]).wait()

        def drain_scatter(b):
            pltpu.make_async_copy(rows[b], acc.at[pl.ds(0, CH)],
                                  ssem[b]).wait()

        def drain_count(b):
            pltpu.make_async_copy(ones, cacc.at[pl.ds(0, CH)],
                                  csem[b]).wait()

        def drain_idx(b):
            pltpu.make_async_copy(dst_hbm.at[pl.ds(0, CH)], dstb[b],
                                  isem[b]).wait()

        for b in range(NBUF):
            issue_gather(b, b)

        def outer(i, carry):
            for b in range(NBUF):
                t = i * NBUF + b
                b1 = (b + NBUF - 1) % NBUF
                b2 = (b + 2) % NBUF
                drain_gather(b)

                @pl.when(t >= NBUF)
                def _():
                    drain_idx(b)

                @pl.when(t >= 1)
                def _():
                    drain_scatter(b1)
                    if with_counts:
                        @pl.when(c == 0)
                        def _():
                            drain_count(b1)

                pltpu.async_copy(rows[b], acc.at[dstb[b]], ssem[b],
                                 add=True)
                if with_counts:
                    @pl.when(c == 0)
                    def _():
                        pltpu.async_copy(ones, cacc.at[dstb[b]], csem[b],
                                         add=True)

                @pl.when(jnp.logical_and(t >= NBUF - 2, t + 2 < NCHUNK))
                def _():
                    pltpu.async_copy(
                        dst_hbm.at[pl.ds(base + (t + 2) * CH, CH)],
                        dstb[b2], isem[b2])
                    issue_gather(t + 2, b2)
            return carry
        lax.fori_loop(0, NMAIN // NBUF, outer, 0)
        for t in range(NMAIN, NCHUNK):
            b = t % NBUF
            b1 = (b + NBUF - 1) % NBUF
            drain_gather(b)
            drain_idx(b)
            drain_scatter(b1)
            if with_counts:
                @pl.when(c == 0)
                def _():
                    drain_count(b1)
            pltpu.async_copy(rows[b], acc.at[dstb[b]], ssem[b], add=True)
            if with_counts:
                @pl.when(c == 0)
                def _():
                    pltpu.async_copy(ones, cacc.at[dstb[b]], csem[b],
                                     add=True)
        drain_scatter((NCHUNK - 1) % NBUF)
        if with_counts:
            @pl.when(c == 0)
            def _():
                drain_count((NCHUNK - 1) % NBUF)
        plsc.subcore_barrier()

        pltpu.sync_copy(acc.at[pl.ds(s * RPZ, RPZ)],
                        s_out.at[c, pl.ds(s * RPZ, RPZ)])
        if with_counts:
            @pl.when(jnp.logical_and(c == 0, s == 0))
            def _():
                pltpu.sync_copy(cacc, cnt_out.at[0])

    return pl.kernel(body, out_type=out_type, mesh=mesh,
                     scratch_types=scratch,
                     compiler_params=pltpu.CompilerParams(
                         use_tc_tiling_on_sc=False))


_segsum_counts = _make_segsum(True)
_segsum_plain = _make_segsum(False)


def kernel(x, edge_index, W1_l, b1_l, W1_r, W2_l, b2_l, W2_r):
    src = edge_index[0]
    dst = edge_index[1]
    zf = jnp.zeros((NP, DH), jnp.float32)
    zc = jnp.zeros((NP,), jnp.float32)

    y1a, y1b, r1 = _tc_pre(x, W1_l, W1_r, b1_l.reshape(1, D))
    s1, cnt = _segsum_counts(y1a, y1b, src, dst, zf, zc)
    y2a, y2b, r2 = _tc_combine_pre(s1, cnt, r1, W2_l, W2_r, b2_l.reshape(1, D))
    (s2,) = _segsum_plain(y2a, y2b, src, dst, zf, zc)
    return _tc_final(s2, cnt, r2)
